# baseline (speedup 1.0000x reference)
import functools

import jax
import jax.numpy as jnp
import numpy as np
from jax import lax
from jax.experimental import pallas as pl
from jax.experimental.pallas import tpu as pltpu
from jax.experimental.pallas import tpu_sc as plsc

TOKENS = 32768
D_MODEL = 768
NUM_CHILDREN = 64
NE = NUM_CHILDREN

BT = 512
NC, NS, L = 2, 16, 16
NW = NC * NS
TOK_PER_W = TOKENS // NW
BLK = 256
FLAT = BLK * NE
NBLK = TOK_PER_W // BLK
NGRP = BLK // L
NEG_INF = float(np.finfo(np.float32).min)


def _mm_body(x_ref, w_ref, b_ref, o_ref):
    o_ref[...] = lax.dot_general(
        x_ref[...], w_ref[...],
        (((1,), (1,)), ((), ())),
        preferred_element_type=jnp.float32,
    ) + b_ref[...]


def _matmul_logits(x, W, b2):
    return pl.pallas_call(
        _mm_body,
        grid=(TOKENS // BT,),
        in_specs=[
            pl.BlockSpec((BT, D_MODEL), lambda i: (i, 0)),
            pl.BlockSpec((NE, D_MODEL), lambda i: (0, 0)),
            pl.BlockSpec((1, NE), lambda i: (0, 0)),
        ],
        out_specs=pl.BlockSpec((BT, NE), lambda i: (i, 0)),
        out_shape=jax.ShapeDtypeStruct((TOKENS, NE), jnp.float32),
        compiler_params=pltpu.CompilerParams(
            dimension_semantics=("arbitrary",),
        ),
    )(x, W, b2)


def _route_body(lg_hbm, z_hbm, out_hbm,
                lg0, lg1, ob0, ob1, idx1, idx2,
                sin0, sin1, sout0, sout1, sz):
    wid = lax.axis_index("s") * NC + lax.axis_index("c")
    base = wid * TOK_PER_W * NE
    lgb = (lg0, lg1)
    obb = (ob0, ob1)
    sin = (sin0, sin1)
    sout = (sout0, sout1)

    zerof = jnp.zeros((L,), jnp.float32)

    in_dma0 = pltpu.async_copy(lg_hbm.at[pl.ds(base, FLAT)], lg0, sin0)
    in_dma1 = pltpu.async_copy(lg_hbm.at[pl.ds(base + FLAT, FLAT)], lg1, sin1)
    pltpu.async_copy(z_hbm, ob0, sz).wait()
    pltpu.async_copy(z_hbm, ob1, sz).wait()

    out_dmas = [None, None]
    for g in range(NBLK):
        bsel = g % 2
        lg = lgb[bsel]
        ob = obb[bsel]
        (in_dma0 if bsel == 0 else in_dma1).wait()
        if out_dmas[bsel] is not None:
            out_dmas[bsel].wait()
            for u in range(NGRP):
                o1 = idx1[pl.ds(bsel * BLK + u * L, L)]
                o2 = idx2[pl.ds(bsel * BLK + u * L, L)]
                plsc.store_scatter(ob, [o1], zerof)
                plsc.store_scatter(ob, [o2], zerof)

        for u in range(NGRP):
            jv0 = (u * L * NE) + lax.iota(jnp.int32, L) * NE

            def step(j, carry):
                jv, m1, i1, m2, i2 = carry
                v = plsc.load_gather(lg, [jv])
                gt1 = v > m1
                gt2 = v > m2
                m2n = jnp.where(gt1, m1, jnp.where(gt2, v, m2))
                i2n = jnp.where(gt1, i1, jnp.where(gt2, jv, i2))
                m1n = jnp.where(gt1, v, m1)
                i1n = jnp.where(gt1, jv, i1)
                return (jv + 1, m1n, i1n, m2n, i2n)

            init = (jv0,
                    jnp.full((L,), NEG_INF, jnp.float32),
                    jv0,
                    jnp.full((L,), NEG_INF, jnp.float32),
                    jv0)
            _, m1, i1, m2, i2 = lax.fori_loop(0, NE, step, init)

            e2 = jnp.exp(m2 - m1)
            w1 = 1.0 / (1.0 + e2)
            w2 = e2 * w1
            plsc.store_scatter(ob, [i1], w1)
            plsc.store_scatter(ob, [i2], w2)
            idx1[pl.ds(bsel * BLK + u * L, L)] = i1
            idx2[pl.ds(bsel * BLK + u * L, L)] = i2

        start = base + g * FLAT
        out_dmas[bsel] = pltpu.async_copy(
            ob, out_hbm.at[pl.ds(start, FLAT)], sout[bsel])
        if g + 2 < NBLK:
            nxt = pltpu.async_copy(
                lg_hbm.at[pl.ds(base + (g + 2) * FLAT, FLAT)], lg, sin[bsel])
            if bsel == 0:
                in_dma0 = nxt
            else:
                in_dma1 = nxt

    out_dmas[0].wait()
    out_dmas[1].wait()


def _route(logits_flat, zblk):
    mesh = plsc.VectorSubcoreMesh(core_axis_name="c", subcore_axis_name="s")
    return pl.kernel(
        _route_body,
        out_type=jax.ShapeDtypeStruct((TOKENS * NE,), jnp.float32),
        mesh=mesh,
        compiler_params=pltpu.CompilerParams(needs_layout_passes=False),
        scratch_types=[
            pltpu.VMEM((FLAT,), jnp.float32),
            pltpu.VMEM((FLAT,), jnp.float32),
            pltpu.VMEM((FLAT,), jnp.float32),
            pltpu.VMEM((FLAT,), jnp.float32),
            pltpu.VMEM((2 * BLK,), jnp.int32),
            pltpu.VMEM((2 * BLK,), jnp.int32),
            pltpu.SemaphoreType.DMA,
            pltpu.SemaphoreType.DMA,
            pltpu.SemaphoreType.DMA,
            pltpu.SemaphoreType.DMA,
            pltpu.SemaphoreType.DMA,
        ],
    )(logits_flat, zblk)


def kernel(payload_tensor, W, b):
    b2 = b.reshape(1, NE)
    logits = _matmul_logits(payload_tensor, W, b2)
    zblk = jnp.zeros((FLAT,), jnp.float32)
    out_flat = _route(logits.reshape(-1), zblk)
    return out_flat.reshape(TOKENS, NE)


# speedup vs baseline: 1.1139x; 1.1139x over previous
import functools

import jax
import jax.numpy as jnp
import numpy as np
from jax import lax
from jax.experimental import pallas as pl
from jax.experimental.pallas import tpu as pltpu
from jax.experimental.pallas import tpu_sc as plsc

TOKENS = 32768
D_MODEL = 768
NUM_CHILDREN = 64
NE = NUM_CHILDREN

BT = 512
NC, NS, L = 2, 16, 16
NW = NC * NS
TOK_PER_W = TOKENS // NW
BLK = 256
FLAT = BLK * NE
NBLK = TOK_PER_W // BLK
NGRP = BLK // L
UNROLL = 8
NEG_INF = float(np.finfo(np.float32).min)


def _mm_body(x_ref, w_ref, b_ref, o_ref):
    o_ref[...] = lax.dot_general(
        x_ref[...], w_ref[...],
        (((1,), (1,)), ((), ())),
        preferred_element_type=jnp.float32,
    ) + b_ref[...]


def _matmul_logits(x, W, b2):
    return pl.pallas_call(
        _mm_body,
        grid=(TOKENS // BT,),
        in_specs=[
            pl.BlockSpec((BT, D_MODEL), lambda i: (i, 0)),
            pl.BlockSpec((NE, D_MODEL), lambda i: (0, 0)),
            pl.BlockSpec((1, NE), lambda i: (0, 0)),
        ],
        out_specs=pl.BlockSpec((BT, NE), lambda i: (i, 0)),
        out_shape=jax.ShapeDtypeStruct((TOKENS, NE), jnp.float32),
        compiler_params=pltpu.CompilerParams(
            dimension_semantics=("arbitrary",),
        ),
    )(x, W, b2)


def _top2_merge(h1, l1, ih1, il1, h2, l2, ih2, il2):
    c = h1 >= h2
    hi = jnp.where(c, h1, h2)
    ihi = jnp.where(c, ih1, ih2)
    m = jnp.where(c, h2, h1)
    im = jnp.where(c, ih2, ih1)
    c2 = l1 >= l2
    lc = jnp.where(c2, l1, l2)
    ilc = jnp.where(c2, il1, il2)
    c3 = m >= lc
    lo = jnp.where(c3, m, lc)
    ilo = jnp.where(c3, im, ilc)
    return hi, lo, ihi, ilo


def _route_body(lg_hbm, z_hbm, out_hbm,
                lg0, lg1, ob0, ob1, ob2, ob3,
                sin0, sin1, so0, so1, so2, so3, sz0, sz1, sz2, sz3):
    wid = lax.axis_index("s") * NC + lax.axis_index("c")
    base = wid * TOK_PER_W * NE
    lgb = (lg0, lg1)
    obb = (ob0, ob1, ob2, ob3)
    sin = (sin0, sin1)
    sout = (so0, so1, so2, so3)
    szs = (sz0, sz1, sz2, sz3)

    in_dmas = [
        pltpu.async_copy(lg_hbm.at[pl.ds(base, FLAT)], lg0, sin0),
        pltpu.async_copy(lg_hbm.at[pl.ds(base + FLAT, FLAT)], lg1, sin1),
    ]
    z_dmas = [pltpu.async_copy(z_hbm, obb[k], szs[k]) for k in range(NBLK)]

    lanes = lax.iota(jnp.int32, L)
    out_dmas = [None] * NBLK
    for g in range(NBLK):
        bsel = g % 2
        lg = lgb[bsel]
        ob = obb[g]
        in_dmas[bsel].wait()
        z_dmas[g].wait()

        def group(u, _):
            jv0 = u * (L * NE) + lanes * NE

            def step(s, carry):
                jv, hv, lv, ih, il = carry
                idxs = [jv + t for t in range(1, UNROLL)]
                idxs = [jv] + idxs
                vals = [plsc.load_gather(lg, [ix]) for ix in idxs]
                t2 = []
                for p in range(UNROLL // 2):
                    a, b = vals[2 * p], vals[2 * p + 1]
                    ia, ib = idxs[2 * p], idxs[2 * p + 1]
                    c = a >= b
                    t2.append((jnp.where(c, a, b), jnp.where(c, b, a),
                               jnp.where(c, ia, ib), jnp.where(c, ib, ia)))
                while len(t2) > 1:
                    nxt = []
                    for p in range(0, len(t2), 2):
                        nxt.append(_top2_merge(*t2[p], *t2[p + 1]))
                    t2 = nxt
                hv, lv, ih, il = _top2_merge(hv, lv, ih, il, *t2[0])
                return (jv + UNROLL, hv, lv, ih, il)

            init = (jv0,
                    jnp.full((L,), NEG_INF, jnp.float32),
                    jnp.full((L,), NEG_INF, jnp.float32),
                    jv0, jv0)
            _, hv, lv, ih, il = lax.fori_loop(0, NE // UNROLL, step, init)

            e2 = jnp.exp(lv - hv)
            w1 = 1.0 / (1.0 + e2)
            w2 = e2 * w1
            plsc.store_scatter(ob, [ih], w1)
            plsc.store_scatter(ob, [il], w2)
            return 0

        lax.fori_loop(0, NGRP, group, 0)

        out_dmas[g] = pltpu.async_copy(
            ob, out_hbm.at[pl.ds(base + g * FLAT, FLAT)], sout[g])
        if g + 2 < NBLK:
            in_dmas[bsel] = pltpu.async_copy(
                lg_hbm.at[pl.ds(base + (g + 2) * FLAT, FLAT)], lg, sin[bsel])

    for g in range(NBLK):
        out_dmas[g].wait()


def _route(logits_flat, zblk):
    mesh = plsc.VectorSubcoreMesh(core_axis_name="c", subcore_axis_name="s")
    return pl.kernel(
        _route_body,
        out_type=jax.ShapeDtypeStruct((TOKENS * NE,), jnp.float32),
        mesh=mesh,
        compiler_params=pltpu.CompilerParams(needs_layout_passes=False),
        scratch_types=[
            pltpu.VMEM((FLAT,), jnp.float32),
            pltpu.VMEM((FLAT,), jnp.float32),
            pltpu.VMEM((FLAT,), jnp.float32),
            pltpu.VMEM((FLAT,), jnp.float32),
            pltpu.VMEM((FLAT,), jnp.float32),
            pltpu.VMEM((FLAT,), jnp.float32),
        ] + [pltpu.SemaphoreType.DMA] * 10,
    )(logits_flat, zblk)


def kernel(payload_tensor, W, b):
    b2 = b.reshape(1, NE)
    logits = _matmul_logits(payload_tensor, W, b2)
    zblk = jnp.zeros((FLAT,), jnp.float32)
    out_flat = _route(logits.reshape(-1), zblk)
    return out_flat.reshape(TOKENS, NE)


# speedup vs baseline: 1.6614x; 1.4915x over previous
import functools

import jax
import jax.numpy as jnp
import numpy as np
from jax import lax
from jax.experimental import pallas as pl
from jax.experimental.pallas import tpu as pltpu
from jax.experimental.pallas import tpu_sc as plsc

TOKENS = 32768
D_MODEL = 768
NUM_CHILDREN = 64
NE = NUM_CHILDREN

BT = 1024
QT = TOKENS // 128
QB = BT // 128
NC, NS, L = 2, 16, 16
NW = NC * NS
TOK_PER_W = TOKENS // NW
BLK = 256
QBLK = BLK // 128
FLAT = BLK * NE
NBLK = TOK_PER_W // BLK
NPAIR = BLK // (2 * L)
UNROLL = 8
NEG_INF = float(np.finfo(np.float32).min)


def _mmT_body(x_ref, w_ref, b_ref, o_ref):
    for qi in range(QB):
        xq = x_ref[pl.ds(qi * 128, 128), :]
        lt = lax.dot_general(
            w_ref[...], xq,
            (((1,), (1,)), ((), ())),
            preferred_element_type=jnp.float32,
        ) + b_ref[...]
        o_ref[qi] = lt


def _matmul_logits_t(x, W, bcol):
    return pl.pallas_call(
        _mmT_body,
        grid=(TOKENS // BT,),
        in_specs=[
            pl.BlockSpec((BT, D_MODEL), lambda i: (i, 0)),
            pl.BlockSpec((NE, D_MODEL), lambda i: (0, 0)),
            pl.BlockSpec((NE, 1), lambda i: (0, 0)),
        ],
        out_specs=pl.BlockSpec((QB, NE, 128), lambda i: (i, 0, 0)),
        out_shape=jax.ShapeDtypeStruct((QT, NE, 128), jnp.float32),
        compiler_params=pltpu.CompilerParams(
            dimension_semantics=("arbitrary",),
        ),
    )(x, W, bcol)


def _merge22(h1, l1, ih1, il1, h2, l2, ih2, il2):
    c1 = h1 >= h2
    hi = jnp.where(c1, h1, h2)
    ihi = jnp.where(c1, ih1, ih2)
    m = jnp.where(c1, h2, h1)
    im = jnp.where(c1, ih2, ih1)
    c2 = l1 >= l2
    lc = jnp.where(c2, l1, l2)
    ilc = jnp.where(c2, il1, il2)
    c3 = m >= lc
    lo = jnp.where(c3, m, lc)
    ilo = jnp.where(c3, im, ilc)
    return hi, lo, ihi, ilo


def _top2_tree8(vals, consts):
    t2 = []
    for p in range(4):
        a, b = vals[2 * p], vals[2 * p + 1]
        c = a >= b
        t2.append((jnp.where(c, a, b), jnp.where(c, b, a),
                   jnp.where(c, consts[2 * p], consts[2 * p + 1]),
                   jnp.where(c, consts[2 * p + 1], consts[2 * p])))
    m1 = _merge22(*t2[0], *t2[1])
    m2 = _merge22(*t2[2], *t2[3])
    return _merge22(*m1, *m2)


def _route_body(lt_hbm, z_hbm, out_hbm,
                kt0, kt1, ob0, ob1, ob2, ob3,
                sin0, sin1, so0, so1, so2, so3, sz0, sz1, sz2, sz3):
    wid = lax.axis_index("s") * NC + lax.axis_index("c")
    tok0 = wid * TOK_PER_W
    base = tok0 * NE
    ktb = (kt0, kt1)
    obb = (ob0, ob1, ob2, ob3)
    sin = (sin0, sin1)
    sout = (so0, so1, so2, so3)
    szs = (sz0, sz1, sz2, sz3)

    q00 = tok0 // 128
    in_dmas = [
        pltpu.async_copy(lt_hbm.at[pl.ds(q00, QBLK), :, :], kt0, sin0),
        pltpu.async_copy(lt_hbm.at[pl.ds(q00 + QBLK, QBLK), :, :], kt1, sin1),
    ]
    z_dmas = [pltpu.async_copy(z_hbm, obb[k], szs[k]) for k in range(NBLK)]

    lanes = lax.iota(jnp.int32, L)
    lanes64 = lanes * NE
    neg = jnp.full((L,), NEG_INF, jnp.float32)
    zero_i = jnp.zeros((L,), jnp.int32)
    consts = [jnp.full((L,), t, jnp.int32) for t in range(UNROLL)]
    out_dmas = [None] * NBLK
    for g in range(NBLK):
        bsel = g % 2
        kt = ktb[bsel]
        ob = obb[g]
        in_dmas[bsel].wait()
        z_dmas[g].wait()

        def pair(up, _):
            ua = up * (2 * L)
            qa = ua // 128
            ca = ua % 128

            def step(s, carry):
                ha, la, iha, ila, hb, lb, ihb, ilb = carry
                s8 = s * UNROLL
                va = [kt[qa, s8 + t, pl.ds(ca, L)] for t in range(UNROLL)]
                vb = [kt[qa, s8 + t, pl.ds(ca + L, L)] for t in range(UNROLL)]
                s8v = jnp.full((L,), 1, jnp.int32) * s8
                tha, tla, tia, tila = _top2_tree8(va, consts)
                thb, tlb, tib, tilb = _top2_tree8(vb, consts)
                ha, la, iha, ila = _merge22(
                    ha, la, iha, ila, tha, tla, tia + s8v, tila + s8v)
                hb, lb, ihb, ilb = _merge22(
                    hb, lb, ihb, ilb, thb, tlb, tib + s8v, tilb + s8v)
                return (ha, la, iha, ila, hb, lb, ihb, ilb)

            init = (neg, neg, zero_i, zero_i, neg, neg, zero_i, zero_i)
            (ha, la, iha, ila,
             hb, lb, ihb, ilb) = lax.fori_loop(0, NE // UNROLL, step, init)

            ta = ua * NE + lanes64
            tb = ta + L * NE
            for (h, l, ih, il, tvec) in ((ha, la, iha, ila, ta),
                                         (hb, lb, ihb, ilb, tb)):
                ex = jnp.exp(l - h)
                w1 = 1.0 / (1.0 + ex)
                w2 = ex * w1
                plsc.store_scatter(ob, [tvec + ih], w1)
                plsc.store_scatter(ob, [tvec + il], w2)
            return 0

        lax.fori_loop(0, NPAIR, pair, 0)

        out_dmas[g] = pltpu.async_copy(
            ob, out_hbm.at[pl.ds(base + g * FLAT, FLAT)], sout[g])
        if g + 2 < NBLK:
            in_dmas[bsel] = pltpu.async_copy(
                lt_hbm.at[pl.ds(q00 + (g + 2) * QBLK, QBLK), :, :],
                kt, sin[bsel])

    for g in range(NBLK):
        out_dmas[g].wait()


def _route(logits_t, zblk):
    mesh = plsc.VectorSubcoreMesh(core_axis_name="c", subcore_axis_name="s")
    return pl.kernel(
        _route_body,
        out_type=jax.ShapeDtypeStruct((TOKENS * NE,), jnp.float32),
        mesh=mesh,
        compiler_params=pltpu.CompilerParams(needs_layout_passes=False),
        scratch_types=[
            pltpu.VMEM((QBLK, NE, 128), jnp.float32),
            pltpu.VMEM((QBLK, NE, 128), jnp.float32),
            pltpu.VMEM((FLAT,), jnp.float32),
            pltpu.VMEM((FLAT,), jnp.float32),
            pltpu.VMEM((FLAT,), jnp.float32),
            pltpu.VMEM((FLAT,), jnp.float32),
        ] + [pltpu.SemaphoreType.DMA] * 10,
    )(logits_t, zblk)


def kernel(payload_tensor, W, b):
    bcol = b.reshape(NE, 1)
    lt = _matmul_logits_t(payload_tensor, W, bcol)
    zblk = jnp.zeros((FLAT,), jnp.float32)
    out_flat = _route(lt, zblk)
    return out_flat.reshape(TOKENS, NE)


# speedup vs baseline: 1.8414x; 1.1083x over previous
import functools

import jax
import jax.numpy as jnp
import numpy as np
from jax import lax
from jax.experimental import pallas as pl
from jax.experimental.pallas import tpu as pltpu
from jax.experimental.pallas import tpu_sc as plsc

TOKENS = 32768
D_MODEL = 768
NUM_CHILDREN = 64
NE = NUM_CHILDREN

BT = 1024
QT = TOKENS // 128
QB = BT // 128
NC, NS, L = 2, 16, 16
NW = NC * NS
TOK_PER_W = TOKENS // NW
BLK = 256
QBLK = BLK // 128
FLAT = BLK * NE
NBLK = TOK_PER_W // BLK
NPAIR = BLK // (2 * L)
UNROLL = 8
NEG_INF = float(np.finfo(np.float32).min)


def _mmT_body(x_ref, w_ref, b_ref, o_ref):
    for qi in range(QB):
        xq = x_ref[pl.ds(qi * 128, 128), :]
        lt = lax.dot_general(
            w_ref[...], xq,
            (((1,), (1,)), ((), ())),
            preferred_element_type=jnp.float32,
        ) + b_ref[...]
        o_ref[qi] = lt


def _matmul_logits_t(x, W, bcol):
    return pl.pallas_call(
        _mmT_body,
        grid=(TOKENS // BT,),
        in_specs=[
            pl.BlockSpec((BT, D_MODEL), lambda i: (i, 0)),
            pl.BlockSpec((NE, D_MODEL), lambda i: (0, 0)),
            pl.BlockSpec((NE, 1), lambda i: (0, 0)),
        ],
        out_specs=pl.BlockSpec((QB, NE, 128), lambda i: (i, 0, 0)),
        out_shape=jax.ShapeDtypeStruct((QT, NE, 128), jnp.float32),
        compiler_params=pltpu.CompilerParams(
            dimension_semantics=("arbitrary",),
        ),
    )(x, W, bcol)


def _merge22(h1, l1, ih1, il1, h2, l2, ih2, il2):
    c1 = h1 >= h2
    hi = jnp.where(c1, h1, h2)
    ihi = jnp.where(c1, ih1, ih2)
    m = jnp.where(c1, h2, h1)
    im = jnp.where(c1, ih2, ih1)
    c2 = l1 >= l2
    lc = jnp.where(c2, l1, l2)
    ilc = jnp.where(c2, il1, il2)
    c3 = m >= lc
    lo = jnp.where(c3, m, lc)
    ilo = jnp.where(c3, im, ilc)
    return hi, lo, ihi, ilo


def _top2_tree8(vals, consts):
    t2 = []
    for p in range(4):
        a, b = vals[2 * p], vals[2 * p + 1]
        c = a >= b
        t2.append((jnp.where(c, a, b), jnp.where(c, b, a),
                   jnp.where(c, consts[2 * p], consts[2 * p + 1]),
                   jnp.where(c, consts[2 * p + 1], consts[2 * p])))
    m1 = _merge22(*t2[0], *t2[1])
    m2 = _merge22(*t2[2], *t2[3])
    return _merge22(*m1, *m2)


def _route_body(lt_hbm, out_hbm,
                kt0, kt1, ob0, ob1, idx1, idx2,
                sin0, sin1, so0, so1):
    wid = lax.axis_index("s") * NC + lax.axis_index("c")
    tok0 = wid * TOK_PER_W
    base = tok0 * NE
    ktb = (kt0, kt1)
    obb = (ob0, ob1)
    sin = (sin0, sin1)
    sout = (so0, so1)

    q00 = tok0 // 128
    in_dmas = [
        pltpu.async_copy(lt_hbm.at[pl.ds(q00, QBLK), :, :], kt0, sin0),
        pltpu.async_copy(lt_hbm.at[pl.ds(q00 + QBLK, QBLK), :, :], kt1, sin1),
    ]

    lanes = lax.iota(jnp.int32, L)
    lanes64 = lanes * NE
    neg = jnp.full((L,), NEG_INF, jnp.float32)
    zero_i = jnp.zeros((L,), jnp.int32)
    zero_f = jnp.zeros((L,), jnp.float32)
    consts = [jnp.full((L,), t, jnp.int32) for t in range(UNROLL)]

    def zloop(i, _):
        for k in range(8):
            ob0[pl.ds(i * 128 + k * L, L)] = zero_f
            ob1[pl.ds(i * 128 + k * L, L)] = zero_f
        return 0
    lax.fori_loop(0, FLAT // 128, zloop, 0)

    out_dmas = [None, None]
    for g in range(NBLK):
        bsel = g % 2
        kt = ktb[bsel]
        ob = obb[bsel]
        in_dmas[bsel].wait()
        if out_dmas[bsel] is not None:
            out_dmas[bsel].wait()

            def rloop(up, _):
                off = bsel * BLK + up * (2 * L)
                for d in range(2):
                    o1 = idx1[pl.ds(off + d * L, L)]
                    o2 = idx2[pl.ds(off + d * L, L)]
                    plsc.store_scatter(ob, [o1], zero_f)
                    plsc.store_scatter(ob, [o2], zero_f)
                return 0
            lax.fori_loop(0, NPAIR, rloop, 0)

        def pair(up, _):
            ua = up * (2 * L)
            qa = ua // 128
            ca = ua % 128

            def step(s, carry):
                ha, la, iha, ila, hb, lb, ihb, ilb = carry
                s8 = s * UNROLL
                va = [kt[qa, s8 + t, pl.ds(ca, L)] for t in range(UNROLL)]
                vb = [kt[qa, s8 + t, pl.ds(ca + L, L)] for t in range(UNROLL)]
                s8v = jnp.full((L,), 1, jnp.int32) * s8
                tha, tla, tia, tila = _top2_tree8(va, consts)
                thb, tlb, tib, tilb = _top2_tree8(vb, consts)
                ha, la, iha, ila = _merge22(
                    ha, la, iha, ila, tha, tla, tia + s8v, tila + s8v)
                hb, lb, ihb, ilb = _merge22(
                    hb, lb, ihb, ilb, thb, tlb, tib + s8v, tilb + s8v)
                return (ha, la, iha, ila, hb, lb, ihb, ilb)

            init = (neg, neg, zero_i, zero_i, neg, neg, zero_i, zero_i)
            (ha, la, iha, ila,
             hb, lb, ihb, ilb) = lax.fori_loop(0, NE // UNROLL, step, init)

            ta = ua * NE + lanes64
            tb = ta + L * NE
            off = bsel * BLK + up * (2 * L)
            for d, (h, l, ih, il, tvec) in enumerate(
                    ((ha, la, iha, ila, ta), (hb, lb, ihb, ilb, tb))):
                ex = jnp.exp(l - h)
                w1 = 1.0 / (1.0 + ex)
                w2 = ex * w1
                s1 = tvec + ih
                s2 = tvec + il
                plsc.store_scatter(ob, [s1], w1)
                plsc.store_scatter(ob, [s2], w2)
                idx1[pl.ds(off + d * L, L)] = s1
                idx2[pl.ds(off + d * L, L)] = s2
            return 0

        lax.fori_loop(0, NPAIR, pair, 0)

        out_dmas[bsel] = pltpu.async_copy(
            ob, out_hbm.at[pl.ds(base + g * FLAT, FLAT)], sout[bsel])
        if g + 2 < NBLK:
            in_dmas[bsel] = pltpu.async_copy(
                lt_hbm.at[pl.ds(q00 + (g + 2) * QBLK, QBLK), :, :],
                kt, sin[bsel])

    out_dmas[0].wait()
    out_dmas[1].wait()


def _route(logits_t):
    mesh = plsc.VectorSubcoreMesh(core_axis_name="c", subcore_axis_name="s")
    return pl.kernel(
        _route_body,
        out_type=jax.ShapeDtypeStruct((TOKENS * NE,), jnp.float32),
        mesh=mesh,
        compiler_params=pltpu.CompilerParams(needs_layout_passes=False),
        scratch_types=[
            pltpu.VMEM((QBLK, NE, 128), jnp.float32),
            pltpu.VMEM((QBLK, NE, 128), jnp.float32),
            pltpu.VMEM((FLAT,), jnp.float32),
            pltpu.VMEM((FLAT,), jnp.float32),
            pltpu.VMEM((2 * BLK,), jnp.int32),
            pltpu.VMEM((2 * BLK,), jnp.int32),
        ] + [pltpu.SemaphoreType.DMA] * 4,
    )(logits_t)


def kernel(payload_tensor, W, b):
    bcol = b.reshape(NE, 1)
    lt = _matmul_logits_t(payload_tensor, W, bcol)
    out_flat = _route(lt)
    return out_flat.reshape(TOKENS, NE)
